# Initial kernel scaffold; baseline (speedup 1.0000x reference)
import jax
import jax.numpy as jnp
from jax.experimental import pallas as pl


def kernel(embedding_matrix, Wq, bq, Wk, bk, sparse_indices):
    raise NotImplementedError("write your pallas kernel here")


# speedup vs baseline: 55.5472x; 55.5472x over previous
import functools

import jax
import jax.numpy as jnp
from jax.experimental import pallas as pl

EMBED = 1024
HEADS = 16
HD = 64
S = 64
N = 2048
BN = 256


def _proj_kernel(x_ref, wq_ref, bq_ref, wk_ref, bk_ref, q_ref, k_ref):
    x = x_ref[...]
    dn = (((1,), (1,)), ((), ()))
    q = jax.lax.dot_general(x, wq_ref[...], dn) + bq_ref[0]
    k = jax.lax.dot_general(x, wk_ref[...], dn) + bk_ref[0]
    q_ref[...] = q[None]
    k_ref[...] = k[None]


def _attn_kernel(q_ref, k_ref, idx_ref, out_ref):
    q = q_ref[0]
    k = k_ref[0]
    idx = idx_ref[0]

    s = jax.lax.dot_general(q, k, (((1,), (1,)), ((), ())))
    s = s * (1.0 / (HD ** 0.5))

    hi = idx >> 7
    lo = idx & 127
    jh = jax.lax.broadcasted_iota(jnp.int32, (BN, S, 16), 2)
    jl = jax.lax.broadcasted_iota(jnp.int32, (BN, S, 128), 2)
    a = (hi[:, :, None] == jh).astype(jnp.bfloat16)
    b = (lo[:, :, None] == jl).astype(jnp.bfloat16)
    cnt = jax.lax.dot_general(
        a, b, (((1,), (1,)), ((0,), (0,))),
        preferred_element_type=jnp.float32)
    mask = cnt.reshape(BN, N) > 0.0
    out_ref[0] = jnp.where(mask, s, 0.0)


@functools.partial(jax.jit, static_argnames=("interpret",))
def _run(x, wq, bq, wk, bk, idx, interpret=False):
    q, k = pl.pallas_call(
        _proj_kernel,
        grid=(HEADS,),
        in_specs=[
            pl.BlockSpec((N, EMBED), lambda h: (0, 0)),
            pl.BlockSpec((HD, EMBED), lambda h: (h, 0)),
            pl.BlockSpec((1, 1, HD), lambda h: (h, 0, 0)),
            pl.BlockSpec((HD, EMBED), lambda h: (h, 0)),
            pl.BlockSpec((1, 1, HD), lambda h: (h, 0, 0)),
        ],
        out_specs=[
            pl.BlockSpec((1, N, HD), lambda h: (h, 0, 0)),
            pl.BlockSpec((1, N, HD), lambda h: (h, 0, 0)),
        ],
        out_shape=[
            jax.ShapeDtypeStruct((HEADS, N, HD), jnp.float32),
            jax.ShapeDtypeStruct((HEADS, N, HD), jnp.float32),
        ],
        interpret=interpret,
    )(x, wq, bq.reshape(HEADS, 1, HD), wk, bk.reshape(HEADS, 1, HD))

    out = pl.pallas_call(
        _attn_kernel,
        grid=(HEADS, N // BN),
        in_specs=[
            pl.BlockSpec((1, BN, HD), lambda h, nb: (h, nb, 0)),
            pl.BlockSpec((1, N, HD), lambda h, nb: (h, 0, 0)),
            pl.BlockSpec((1, BN, S), lambda h, nb: (h, nb, 0)),
        ],
        out_specs=pl.BlockSpec((1, BN, N), lambda h, nb: (h, nb, 0)),
        out_shape=jax.ShapeDtypeStruct((HEADS, N, N), jnp.float32),
        interpret=interpret,
    )(q, k, idx)
    return out


def kernel(embedding_matrix, Wq, bq, Wk, bk, sparse_indices):
    return _run(embedding_matrix, Wq, bq, Wk, bk, sparse_indices)


# speedup vs baseline: 65.2478x; 1.1746x over previous
import functools

import jax
import jax.numpy as jnp
from jax.experimental import pallas as pl

EMBED = 1024
HEADS = 16
HD = 64
S = 64
N = 2048
BN = 256


def _proj_kernel(x_ref, wq_ref, bq_ref, wk_ref, bk_ref, q_ref, k_ref):
    x = x_ref[...]
    dn = (((1,), (1,)), ((), ()))
    q = jax.lax.dot_general(x, wq_ref[...], dn) + bq_ref[0]
    k = jax.lax.dot_general(x, wk_ref[...], dn) + bk_ref[0]
    q_ref[...] = q[None]
    k_ref[...] = k[None]


def _attn_kernel(q_ref, k_ref, idx_ref, out_ref):
    q = q_ref[0]
    k = k_ref[0]
    idx = idx_ref[0]

    s = jax.lax.dot_general(q, k, (((1,), (1,)), ((), ())))

    hi = idx >> 7
    lo = idx & 127
    jh = jax.lax.broadcasted_iota(jnp.int32, (BN, 16, S), 1)
    jl = jax.lax.broadcasted_iota(jnp.int32, (BN, 128, S), 1)
    a = (hi[:, None, :] == jh).astype(jnp.bfloat16)
    b = (lo[:, None, :] == jl).astype(jnp.bfloat16)
    cnt = jax.lax.dot_general(
        a, b, (((2,), (2,)), ((0,), (0,))),
        preferred_element_type=jnp.float32)
    mask = cnt.reshape(BN, N) > 0.0
    out_ref[0] = jnp.where(mask, s, 0.0)


@functools.partial(jax.jit, static_argnames=("interpret",))
def _run(x, wq, bq, wk, bk, idx, interpret=False):
    q, k = pl.pallas_call(
        _proj_kernel,
        grid=(HEADS,),
        in_specs=[
            pl.BlockSpec((N, EMBED), lambda h: (0, 0)),
            pl.BlockSpec((HD, EMBED), lambda h: (h, 0)),
            pl.BlockSpec((1, 1, HD), lambda h: (h, 0, 0)),
            pl.BlockSpec((HD, EMBED), lambda h: (h, 0)),
            pl.BlockSpec((1, 1, HD), lambda h: (h, 0, 0)),
        ],
        out_specs=[
            pl.BlockSpec((1, N, HD), lambda h: (h, 0, 0)),
            pl.BlockSpec((1, N, HD), lambda h: (h, 0, 0)),
        ],
        out_shape=[
            jax.ShapeDtypeStruct((HEADS, N, HD), jnp.float32),
            jax.ShapeDtypeStruct((HEADS, N, HD), jnp.float32),
        ],
        interpret=interpret,
    )(x, wq, bq.reshape(HEADS, 1, HD), wk, bk.reshape(HEADS, 1, HD))

    out = pl.pallas_call(
        _attn_kernel,
        grid=(HEADS, N // BN),
        in_specs=[
            pl.BlockSpec((1, BN, HD), lambda h, nb: (h, nb, 0)),
            pl.BlockSpec((1, N, HD), lambda h, nb: (h, 0, 0)),
            pl.BlockSpec((1, BN, S), lambda h, nb: (h, nb, 0)),
        ],
        out_specs=pl.BlockSpec((1, BN, N), lambda h, nb: (h, nb, 0)),
        out_shape=jax.ShapeDtypeStruct((HEADS, N, N), jnp.float32),
        interpret=interpret,
    )(q, k, idx)
    return out


def kernel(embedding_matrix, Wq, bq, Wk, bk, sparse_indices):
    return _run(embedding_matrix, Wq * 0.125, bq * 0.125, Wk, bk,
                sparse_indices)
